# baseline (speedup 1.0000x reference)
import dataclasses
import functools

import jax
import jax.numpy as jnp
from jax.experimental import pallas as pl
from jax.experimental.pallas import tpu as pltpu
from jax.experimental.pallas import tpu_sc as plsc

DIM = 128
BATCH = 16384
NEG = 20
_KBLK = 1 + NEG
_KPAD = 32

_NW = 32
_EPB = BATCH // _NW
_EW = 8
_NR = _EW * NEG
_WIN = _EPB // _EW
_NCH = DIM // 16
_NBUF = 4


def _sc_dots(embeddings, context_embeddings, tgt_idx, ctx_idx, neg_idx):
    mesh = plsc.VectorSubcoreMesh(core_axis_name="c", subcore_axis_name="s")
    cp = pltpu.CompilerParams()
    if "needs_layout_passes" in pltpu.CompilerParams.__dataclass_fields__:
        cp = dataclasses.replace(cp, needs_layout_passes=False)

    @functools.partial(
        pl.kernel,
        out_type=jax.ShapeDtypeStruct((BATCH * _KPAD,), jnp.float32),
        mesh=mesh,
        compiler_params=cp,
        scratch_types=(
            [pltpu.VMEM((_EPB,), jnp.int32),
             pltpu.VMEM((_EPB,), jnp.int32),
             pltpu.VMEM((_EPB * NEG,), jnp.int32),
             pltpu.VMEM((_EPB * _KPAD,), jnp.float32)]
            + [pltpu.VMEM((_NR, DIM), jnp.float32)] * _NBUF
            + [pltpu.VMEM((_EW, DIM), jnp.float32)] * _NBUF
            + [pltpu.VMEM((_EW, DIM), jnp.float32)] * _NBUF
            + [pltpu.SemaphoreType.DMA] * _NBUF
        ),
    )
    def dots_kernel(emb_hbm, cemb_hbm, ti_hbm, ci_hbm, ni_hbm, out_hbm,
                    ti_v, ci_v, ni_v, dots_v, *scr):
        nbb = scr[:_NBUF]
        cbb = scr[_NBUF:2 * _NBUF]
        tbb = scr[2 * _NBUF:3 * _NBUF]
        sems = scr[3 * _NBUF:]
        wid = jax.lax.axis_index("s") * 2 + jax.lax.axis_index("c")
        el_base = wid * _EPB

        pltpu.sync_copy(ti_hbm.at[pl.ds(el_base, _EPB)], ti_v)
        pltpu.sync_copy(ci_hbm.at[pl.ds(el_base, _EPB)], ci_v)
        pltpu.sync_copy(ni_hbm.at[pl.ds(el_base * NEG, _EPB * NEG)], ni_v)

        def start_gather(w, b):
            off = w * _NR
            pltpu.async_copy(cemb_hbm.at[ni_v.at[pl.ds(off, 128)]],
                             nbb[b].at[pl.ds(0, 128)], sems[b])
            pltpu.async_copy(cemb_hbm.at[ni_v.at[pl.ds(off + 128, _NR - 128)]],
                             nbb[b].at[pl.ds(128, _NR - 128)], sems[b])
            pltpu.async_copy(cemb_hbm.at[ci_v.at[pl.ds(w * _EW, _EW)]],
                             cbb[b], sems[b])
            pltpu.async_copy(emb_hbm.at[ti_v.at[pl.ds(w * _EW, _EW)]],
                             tbb[b], sems[b])

        def wait_gather(b):
            pltpu.make_async_copy(cemb_hbm.at[ni_v.at[pl.ds(0, 128)]],
                                  nbb[b].at[pl.ds(0, 128)], sems[b]).wait()
            pltpu.make_async_copy(cemb_hbm.at[ni_v.at[pl.ds(0, _NR - 128)]],
                                  nbb[b].at[pl.ds(128, _NR - 128)],
                                  sems[b]).wait()
            pltpu.make_async_copy(cemb_hbm.at[ci_v.at[pl.ds(0, _EW)]],
                                  cbb[b], sems[b]).wait()
            pltpu.make_async_copy(emb_hbm.at[ti_v.at[pl.ds(0, _EW)]],
                                  tbb[b], sems[b]).wait()

        lanes = jax.lax.iota(jnp.int32, 16)

        def compute(w, b):
            @pl.loop(0, _EW, step=2)
            def _(e0):
                for de in range(2):
                    e = e0 + de
                    t = [tbb[b][e, pl.ds(j * 16, 16)] for j in range(_NCH)]
                    col = w * _EW + e
                    v = [jnp.zeros((16,), jnp.float32) for _ in range(2)]
                    for k in range(_KBLK):
                        src = cbb[b] if k == 0 else nbb[b]
                        row = e if k == 0 else e * NEG + (k - 1)
                        acc = t[0] * src[row, pl.ds(0, 16)]
                        for j in range(1, _NCH):
                            acc = acc + t[j] * src[row, pl.ds(j * 16, 16)]
                        s = jnp.broadcast_to(jnp.sum(acc), (16,))
                        h = k // 16
                        v[h] = jnp.where(lanes == (k % 16), s, v[h])
                    dots_v[pl.ds(col * _KPAD, 16)] = v[0]
                    dots_v[pl.ds(col * _KPAD + 16, 16)] = v[1]

        for b in range(_NBUF):
            start_gather(b, b)

        @pl.loop(0, _WIN - _NBUF, step=_NBUF)
        def _(w0):
            for b in range(_NBUF):
                wait_gather(b)
                compute(w0 + b, b)
                start_gather(w0 + _NBUF + b, b)

        for b in range(_NBUF):
            wait_gather(b)
            compute(_WIN - _NBUF + b, b)

        pltpu.sync_copy(dots_v,
                        out_hbm.at[pl.ds(el_base * _KPAD, _EPB * _KPAD)])

    return dots_kernel(embeddings, context_embeddings,
                       tgt_idx, ctx_idx, neg_idx)


_FIN_R = BATCH * _KPAD // DIM


def _finish_body(d_ref, out_ref):
    dm = d_ref[...]
    col = jax.lax.broadcasted_iota(jnp.int32, (_FIN_R, DIM), 1) % _KPAD
    s = jnp.where(col == 0, jax.nn.log_sigmoid(dm), jax.nn.log_sigmoid(-dm))
    s = jnp.where(col < _KBLK, s, 0.0)
    out_ref[...] = (jnp.sum(s) * (-1.0 / BATCH)).reshape(1, 1)


def kernel(target, context, negative_samples, embeddings, context_embeddings):
    tgt = target.astype(jnp.int32)
    ctx = context.astype(jnp.int32)
    neg = negative_samples.astype(jnp.int32).reshape(-1)

    dots = _sc_dots(embeddings, context_embeddings, tgt, ctx, neg)

    loss = pl.pallas_call(
        _finish_body,
        in_specs=[pl.BlockSpec((_FIN_R, DIM), lambda: (0, 0))],
        out_specs=pl.BlockSpec((1, 1), lambda: (0, 0)),
        out_shape=jax.ShapeDtypeStruct((1, 1), jnp.float32),
    )(dots.reshape(_FIN_R, DIM))
    return loss[0, 0]
